# Initial kernel scaffold; baseline (speedup 1.0000x reference)
import jax
import jax.numpy as jnp
from jax.experimental import pallas as pl


def kernel(midi_events, event_types, note_table, pos_enc, special_dur, pos_w, pos_b, vel_w, vel_b, type_table, ln_g, ln_b):
    raise NotImplementedError("write your pallas kernel here")


# speedup vs baseline: 6.4890x; 6.4890x over previous
import functools

import jax
import jax.numpy as jnp
from jax import lax
from jax.experimental import pallas as pl
from jax.experimental.pallas import tpu as pltpu
from jax.experimental.pallas import tpu_sc as plsc

_NC, _NS, _L = 2, 16, 16
_NW = _NC * _NS
_B, _SEQ, _D = 1024, 200, 64
_N = _B * _SEQ
_PER_W = _N // _NW
_C = 128
_CHUNKS = _PER_W // _C


def _tables_body(pos90_ref, note_ref, posw_ref, posb_ref, sdur_ref, ttab_ref,
                 velw_ref, velb_ref, t1_ref, t2_ref):
    pos90 = pos90_ref[...]
    d2 = lax.dot_general(pos90, posw_ref[...],
                         (((1,), (1,)), ((), ())),
                         preferred_element_type=jnp.float32)
    d2 = d2 + posb_ref[...]
    row = lax.broadcasted_iota(jnp.int32, (90, 1), 0)
    d2 = jnp.where(row == 0, sdur_ref[...], d2)
    t1_ref[...] = pos90[:, None, :] + note_ref[...][None, :, :]
    vel = row.astype(jnp.float32) / 10.0
    velrows = vel * velw_ref[...] + velb_ref[...]
    t2_ref[...] = (ttab_ref[...][:, None, None, :]
                   + d2[None, :, None, :]
                   + velrows[None, None, :, :])


def _idx_body(midi_ref, et_ref, i1_ref, i2_ref):
    x = midi_ref[0]
    a = x[0:1, :]
    n = x[1:2, :]
    d = x[2:3, :]
    v = x[3:4, :]
    t = et_ref[0]
    i1_ref[0] = a * 90 + n
    i2_ref[0] = t * 8100 + d * 90 + v


_IDX_BLK = 8192
_IDX_ROWS = _N // _IDX_BLK


def _build_idx(me3, et3):
    grid = (_IDX_ROWS,)
    return pl.pallas_call(
        _idx_body,
        grid=grid,
        in_specs=[
            pl.BlockSpec((1, 4, _IDX_BLK), lambda g: (g, 0, 0)),
            pl.BlockSpec((1, 1, _IDX_BLK), lambda g: (g, 0, 0)),
        ],
        out_specs=[
            pl.BlockSpec((1, 1, _IDX_BLK), lambda g: (g, 0, 0)),
            pl.BlockSpec((1, 1, _IDX_BLK), lambda g: (g, 0, 0)),
        ],
        out_shape=[
            jax.ShapeDtypeStruct((_IDX_ROWS, 1, _IDX_BLK), jnp.int32),
            jax.ShapeDtypeStruct((_IDX_ROWS, 1, _IDX_BLK), jnp.int32),
        ],
    )(me3, et3)


def _build_tables(pos90, note_table, pos_w, pos_b, special_dur, type_table,
                  vel_wT, vel_b):
    return pl.pallas_call(
        _tables_body,
        out_shape=[
            jax.ShapeDtypeStruct((90, 90, _D), jnp.float32),
            jax.ShapeDtypeStruct((2, 90, 90, _D), jnp.float32),
        ],
    )(pos90, note_table, pos_w, pos_b.reshape(1, _D), special_dur,
      type_table, vel_wT, vel_b.reshape(1, _D))


def _rsqrt(x):
    i = lax.bitcast_convert_type(x, jnp.int32)
    i = jnp.int32(0x5F3759DF) - (i >> 1)
    y = lax.bitcast_convert_type(i, jnp.float32)
    for _ in range(3):
        y = y * (1.5 - 0.5 * x * y * y)
    return y


def _sc_body(i1_hbm, i2_hbm, t1_hbm, t2_hbm, out_hbm,
             i1_v, i2_v, b1_v, b2_v, sem1, sem2):
    wid = lax.axis_index("s") * _NC + lax.axis_index("c")

    def chunk(ci, carry):
        base = wid * _PER_W + ci * _C
        pltpu.sync_copy(i1_hbm.at[pl.ds(base, _C)], i1_v)
        pltpu.sync_copy(i2_hbm.at[pl.ds(base, _C)], i2_v)

        cp1 = pltpu.async_copy(t1_hbm.at[i1_v], b1_v, sem1)
        cp2 = pltpu.async_copy(t2_hbm.at[i2_v], b2_v, sem2)
        cp1.wait()
        cp2.wait()

        def tok(t, _):
            for k in range(4):
                b1_v[t, pl.ds(16 * k, 16)] = (b1_v[t, pl.ds(16 * k, 16)]
                                              + b2_v[t, pl.ds(16 * k, 16)])
            return 0

        lax.fori_loop(0, _C, tok, 0)
        pltpu.sync_copy(b1_v, out_hbm.at[pl.ds(base, _C)])
        return carry

    lax.fori_loop(0, _CHUNKS, chunk, 0)


@functools.cache
def _sc_main():
    return pl.kernel(
        _sc_body,
        out_type=jax.ShapeDtypeStruct((_N, _D), jnp.float32),
        mesh=plsc.VectorSubcoreMesh(core_axis_name="c", subcore_axis_name="s",
                                    num_cores=_NC, num_subcores=_NS),
        compiler_params=pltpu.CompilerParams(use_tc_tiling_on_sc=False),
        scratch_types=[
            pltpu.VMEM((_C,), jnp.int32),
            pltpu.VMEM((_C,), jnp.int32),
            pltpu.VMEM((_C, _D), jnp.float32),
            pltpu.VMEM((_C, _D), jnp.float32),
            pltpu.SemaphoreType.DMA,
            pltpu.SemaphoreType.DMA,
        ],
    )


_LN_BLK = 4096


def _ln_body(x_ref, g_ref, b_ref, o_ref):
    x = x_ref[...]
    mean = jnp.mean(x, axis=-1, keepdims=True)
    cent = x - mean
    var = jnp.mean(cent * cent, axis=-1, keepdims=True)
    o_ref[...] = cent * lax.rsqrt(var + 1e-5) * g_ref[...] + b_ref[...]


def _layernorm(x, ln_g, ln_b):
    grid = (_N // _LN_BLK,)
    return pl.pallas_call(
        _ln_body,
        grid=grid,
        in_specs=[
            pl.BlockSpec((_LN_BLK, _D), lambda g: (g, 0)),
            pl.BlockSpec((1, _D), lambda g: (0, 0)),
            pl.BlockSpec((1, _D), lambda g: (0, 0)),
        ],
        out_specs=pl.BlockSpec((_LN_BLK, _D), lambda g: (g, 0)),
        out_shape=jax.ShapeDtypeStruct((_N, _D), jnp.float32),
    )(x, ln_g.reshape(1, _D), ln_b.reshape(1, _D))


def kernel(midi_events, event_types, note_table, pos_enc, special_dur, pos_w,
           pos_b, vel_w, vel_b, type_table, ln_g, ln_b):
    pos90 = pos_enc[:90]
    t1_3d, t2_4d = _build_tables(pos90, note_table, pos_w, pos_b, special_dur,
                                 type_table, vel_w.reshape(1, _D), vel_b)
    t1 = t1_3d.reshape(90 * 90, _D)
    t2 = t2_4d.reshape(2 * 90 * 90, _D)
    me3 = midi_events.reshape(_IDX_ROWS, _IDX_BLK, 4).transpose(0, 2, 1)
    et3 = event_types.reshape(_IDX_ROWS, 1, _IDX_BLK)
    i1_3, i2_3 = _build_idx(me3, et3)
    i1 = i1_3.reshape(_N)
    i2 = i2_3.reshape(_N)
    comb = _sc_main()(i1, i2, t1, t2)
    out = _layernorm(comb, ln_g, ln_b)
    return out.reshape(_B, _SEQ, _D)


# speedup vs baseline: 8.0815x; 1.2454x over previous
import functools

import jax
import jax.numpy as jnp
from jax import lax
from jax.experimental import pallas as pl
from jax.experimental.pallas import tpu as pltpu
from jax.experimental.pallas import tpu_sc as plsc

_NC, _NS, _L = 2, 16, 16
_NW = _NC * _NS
_B, _SEQ, _D = 1024, 200, 64
_N = _B * _SEQ
_PER_W = _N // _NW
_C = 128
_CHUNKS = _PER_W // _C


def _tables_body(pos90_ref, note_ref, posw_ref, posb_ref, sdur_ref, ttab_ref,
                 velw_ref, velb_ref, t1_ref, t2_ref):
    pos90 = pos90_ref[...]
    d2 = lax.dot_general(pos90, posw_ref[...],
                         (((1,), (1,)), ((), ())),
                         preferred_element_type=jnp.float32)
    d2 = d2 + posb_ref[...]
    row = lax.broadcasted_iota(jnp.int32, (90, 1), 0)
    d2 = jnp.where(row == 0, sdur_ref[...], d2)
    t1_ref[...] = pos90[:, None, :] + note_ref[...][None, :, :]
    vel = row.astype(jnp.float32) / 10.0
    velrows = vel * velw_ref[...] + velb_ref[...]
    t2_ref[...] = (ttab_ref[...][:, None, None, :]
                   + d2[None, :, None, :]
                   + velrows[None, None, :, :])


def _build_tables(pos90, note_table, pos_w, pos_b, special_dur, type_table,
                  vel_wT, vel_b):
    return pl.pallas_call(
        _tables_body,
        out_shape=[
            jax.ShapeDtypeStruct((90, 90, _D), jnp.float32),
            jax.ShapeDtypeStruct((2, 90, 90, _D), jnp.float32),
        ],
    )(pos90, note_table, pos_w, pos_b.reshape(1, _D), special_dur,
      type_table, vel_wT, vel_b.reshape(1, _D))


_IDX_BLK = 8192
_IDX_ROWS = _N // _IDX_BLK


def _idx_body(a_ref, n_ref, d_ref, v_ref, t_ref, i1_ref, i2_ref):
    i1_ref[...] = a_ref[...] * 90 + n_ref[...]
    i2_ref[...] = (t_ref[...] * 90 + d_ref[...]) * 90 + v_ref[...]


def _build_idx(a3, n3, d3, v3, t3):
    spec = pl.BlockSpec((1, 1, _IDX_BLK), lambda g: (g, 0, 0))
    return pl.pallas_call(
        _idx_body,
        grid=(_IDX_ROWS,),
        in_specs=[spec] * 5,
        out_specs=[spec, spec],
        out_shape=[
            jax.ShapeDtypeStruct((_IDX_ROWS, 1, _IDX_BLK), jnp.int32),
            jax.ShapeDtypeStruct((_IDX_ROWS, 1, _IDX_BLK), jnp.int32),
        ],
    )(a3, n3, d3, v3, t3)


def _sc_body(i1_hbm, i2_hbm, t1_hbm, t2_hbm, out_hbm,
             i1_v, i2_v, b1_a, b1_b, b2_a, b2_b, o_a, o_b,
             gsem_a, gsem_b, osem_a, osem_b):
    wid = lax.axis_index("s") * _NC + lax.axis_index("c")
    base0 = wid * _PER_W
    pltpu.sync_copy(i1_hbm.at[pl.ds(base0, _PER_W)], i1_v)
    pltpu.sync_copy(i2_hbm.at[pl.ds(base0, _PER_W)], i2_v)

    sides = (
        (b1_a, b2_a, o_a, gsem_a, osem_a),
        (b1_b, b2_b, o_b, gsem_b, osem_b),
    )

    def fire_gathers(side, ci):
        b1_v, b2_v, _, gsem, _ = sides[side]
        pltpu.async_copy(t1_hbm.at[i1_v.at[pl.ds(ci * _C, _C)]], b1_v, gsem)
        pltpu.async_copy(t2_hbm.at[i2_v.at[pl.ds(ci * _C, _C)]], b2_v, gsem)

    def process(side, oi):
        b1_v, b2_v, o_v, gsem, osem = sides[side]
        ci = 2 * oi + side
        head = i1_v.at[pl.ds(0, _C)]
        pltpu.make_async_copy(t1_hbm.at[head], b1_v, gsem).wait()
        pltpu.make_async_copy(t2_hbm.at[head], b2_v, gsem).wait()
        @pl.when(ci >= 2)
        def _():
            pltpu.make_async_copy(o_v, out_hbm.at[pl.ds(base0, _C)],
                                  osem).wait()

        def tok(ti, carry):
            t0 = ti * 8
            for u in range(8):
                for k in range(4):
                    sl = pl.ds(_L * k, _L)
                    o_v[t0 + u, sl] = b1_v[t0 + u, sl] + b2_v[t0 + u, sl]
            return carry

        lax.fori_loop(0, _C // 8, tok, 0)
        pltpu.async_copy(o_v, out_hbm.at[pl.ds(base0 + ci * _C, _C)], osem)

        @pl.when(ci + 2 < _CHUNKS)
        def _():
            fire_gathers(side, ci + 2)

    fire_gathers(0, 0)
    fire_gathers(1, 1)

    def outer(oi, carry):
        process(0, oi)
        process(1, oi)
        return carry

    lax.fori_loop(0, _CHUNKS // 2, outer, 0)
    for _, _, o_v, _, osem in sides:
        pltpu.make_async_copy(o_v, out_hbm.at[pl.ds(base0, _C)], osem).wait()


@functools.cache
def _sc_main():
    return pl.kernel(
        _sc_body,
        out_type=jax.ShapeDtypeStruct((_N, _D), jnp.float32),
        mesh=plsc.VectorSubcoreMesh(core_axis_name="c", subcore_axis_name="s",
                                    num_cores=_NC, num_subcores=_NS),
        compiler_params=pltpu.CompilerParams(use_tc_tiling_on_sc=False),
        scratch_types=[
            pltpu.VMEM((_PER_W,), jnp.int32),
            pltpu.VMEM((_PER_W,), jnp.int32),
            pltpu.VMEM((_C, _D), jnp.float32),
            pltpu.VMEM((_C, _D), jnp.float32),
            pltpu.VMEM((_C, _D), jnp.float32),
            pltpu.VMEM((_C, _D), jnp.float32),
            pltpu.VMEM((_C, _D), jnp.float32),
            pltpu.VMEM((_C, _D), jnp.float32),
            pltpu.SemaphoreType.DMA,
            pltpu.SemaphoreType.DMA,
            pltpu.SemaphoreType.DMA,
            pltpu.SemaphoreType.DMA,
        ],
    )


_LN_BLK = 4096


def _ln_body(x_ref, g_ref, b_ref, o_ref):
    x = x_ref[...]
    mean = jnp.mean(x, axis=-1, keepdims=True)
    cent = x - mean
    var = jnp.mean(cent * cent, axis=-1, keepdims=True)
    o_ref[...] = cent * lax.rsqrt(var + 1e-5) * g_ref[...] + b_ref[...]


def _layernorm(x, ln_g, ln_b):
    grid = (_N // _LN_BLK,)
    return pl.pallas_call(
        _ln_body,
        grid=grid,
        in_specs=[
            pl.BlockSpec((_LN_BLK, _D), lambda g: (g, 0)),
            pl.BlockSpec((1, _D), lambda g: (0, 0)),
            pl.BlockSpec((1, _D), lambda g: (0, 0)),
        ],
        out_specs=pl.BlockSpec((_LN_BLK, _D), lambda g: (g, 0)),
        out_shape=jax.ShapeDtypeStruct((_N, _D), jnp.float32),
    )(x, ln_g.reshape(1, _D), ln_b.reshape(1, _D))


def kernel(midi_events, event_types, note_table, pos_enc, special_dur, pos_w,
           pos_b, vel_w, vel_b, type_table, ln_g, ln_b):
    pos90 = pos_enc[:90]
    t1_3d, t2_4d = _build_tables(pos90, note_table, pos_w, pos_b, special_dur,
                                 type_table, vel_w.reshape(1, _D), vel_b)
    t1 = t1_3d.reshape(90 * 90, _D)
    t2 = t2_4d.reshape(2 * 90 * 90, _D)
    me = midi_events.reshape(_N, 4)
    shp = (_IDX_ROWS, 1, _IDX_BLK)
    a3 = me[:, 0].reshape(shp)
    n3 = me[:, 1].reshape(shp)
    d3 = me[:, 2].reshape(shp)
    v3 = me[:, 3].reshape(shp)
    t3 = event_types.reshape(shp)
    i1_3, i2_3 = _build_idx(a3, n3, d3, v3, t3)
    i1 = i1_3.reshape(_N)
    i2 = i2_3.reshape(_N)
    comb = _sc_main()(i1, i2, t1, t2)
    out = _layernorm(comb, ln_g, ln_b)
    return out.reshape(_B, _SEQ, _D)


# speedup vs baseline: 8.1692x; 1.0108x over previous
import functools

import jax
import jax.numpy as jnp
from jax import lax
from jax.experimental import pallas as pl
from jax.experimental.pallas import tpu as pltpu
from jax.experimental.pallas import tpu_sc as plsc

_NC, _NS, _L = 2, 16, 16
_NW = _NC * _NS
_B, _SEQ, _D = 1024, 200, 64
_N = _B * _SEQ
_PER_W = _N // _NW
_C = 128
_CHUNKS = _PER_W // _C


def _tables_body(pos90_ref, note_ref, posw_ref, posb_ref, sdur_ref, ttab_ref,
                 velw_ref, velb_ref, t1_ref, t2_ref):
    pos90 = pos90_ref[...]
    d2 = lax.dot_general(pos90, posw_ref[...],
                         (((1,), (1,)), ((), ())),
                         preferred_element_type=jnp.float32)
    d2 = d2 + posb_ref[...]
    row = lax.broadcasted_iota(jnp.int32, (90, 1), 0)
    d2 = jnp.where(row == 0, sdur_ref[...], d2)
    t1_ref[...] = pos90[:, None, :] + note_ref[...][None, :, :]
    vel = row.astype(jnp.float32) / 10.0
    velrows = vel * velw_ref[...] + velb_ref[...]
    t2_ref[...] = (ttab_ref[...][:, None, None, :]
                   + d2[None, :, None, :]
                   + velrows[None, None, :, :])


def _build_tables(pos90, note_table, pos_w, pos_b, special_dur, type_table,
                  vel_wT, vel_b):
    return pl.pallas_call(
        _tables_body,
        out_shape=[
            jax.ShapeDtypeStruct((90, 90, _D), jnp.float32),
            jax.ShapeDtypeStruct((2, 90, 90, _D), jnp.float32),
        ],
    )(pos90, note_table, pos_w, pos_b.reshape(1, _D), special_dur,
      type_table, vel_wT, vel_b.reshape(1, _D))


_IDX_BLK = 8192
_IDX_ROWS = _N // _IDX_BLK


def _idx_body(me_ref, et_ref, i1_ref, i2_ref):
    x = me_ref[...]
    t = et_ref[0]
    i1_ref[...] = x[0] * 90 + x[1]
    i2_ref[...] = (t * 90 + x[2]) * 90 + x[3]


def _build_idx(meT, et3):
    return pl.pallas_call(
        _idx_body,
        out_shape=[
            jax.ShapeDtypeStruct((_IDX_ROWS, _IDX_BLK), jnp.int32),
            jax.ShapeDtypeStruct((_IDX_ROWS, _IDX_BLK), jnp.int32),
        ],
    )(meT, et3)


def _sc_body(i1_hbm, i2_hbm, t1_hbm, t2_hbm, out_hbm,
             i1_v, i2_v, b1_a, b1_b, b2_a, b2_b, o_a, o_b,
             gsem_a, gsem_b, osem_a, osem_b):
    wid = lax.axis_index("s") * _NC + lax.axis_index("c")
    base0 = wid * _PER_W
    pltpu.sync_copy(i1_hbm.at[pl.ds(base0, _PER_W)], i1_v)
    pltpu.sync_copy(i2_hbm.at[pl.ds(base0, _PER_W)], i2_v)

    sides = (
        (b1_a, b2_a, o_a, gsem_a, osem_a),
        (b1_b, b2_b, o_b, gsem_b, osem_b),
    )

    def fire_gathers(side, ci):
        b1_v, b2_v, _, gsem, _ = sides[side]
        pltpu.async_copy(t1_hbm.at[i1_v.at[pl.ds(ci * _C, _C)]], b1_v, gsem)
        pltpu.async_copy(t2_hbm.at[i2_v.at[pl.ds(ci * _C, _C)]], b2_v, gsem)

    def process(side, oi):
        b1_v, b2_v, o_v, gsem, osem = sides[side]
        ci = 2 * oi + side
        head = i1_v.at[pl.ds(0, _C)]
        pltpu.make_async_copy(t1_hbm.at[head], b1_v, gsem).wait()
        pltpu.make_async_copy(t2_hbm.at[head], b2_v, gsem).wait()
        @pl.when(ci >= 2)
        def _():
            pltpu.make_async_copy(o_v, out_hbm.at[pl.ds(base0, _C)],
                                  osem).wait()

        def tok(ti, carry):
            t0 = ti * 8
            for u in range(8):
                for k in range(4):
                    sl = pl.ds(_L * k, _L)
                    o_v[t0 + u, sl] = b1_v[t0 + u, sl] + b2_v[t0 + u, sl]
            return carry

        lax.fori_loop(0, _C // 8, tok, 0)
        pltpu.async_copy(o_v, out_hbm.at[pl.ds(base0 + ci * _C, _C)], osem)

        @pl.when(ci + 2 < _CHUNKS)
        def _():
            fire_gathers(side, ci + 2)

    fire_gathers(0, 0)
    fire_gathers(1, 1)

    def outer(oi, carry):
        process(0, oi)
        process(1, oi)
        return carry

    lax.fori_loop(0, _CHUNKS // 2, outer, 0)
    for _, _, o_v, _, osem in sides:
        pltpu.make_async_copy(o_v, out_hbm.at[pl.ds(base0, _C)], osem).wait()


@functools.cache
def _sc_main():
    return pl.kernel(
        _sc_body,
        out_type=jax.ShapeDtypeStruct((_N, _D), jnp.float32),
        mesh=plsc.VectorSubcoreMesh(core_axis_name="c", subcore_axis_name="s",
                                    num_cores=_NC, num_subcores=_NS),
        compiler_params=pltpu.CompilerParams(use_tc_tiling_on_sc=False),
        scratch_types=[
            pltpu.VMEM((_PER_W,), jnp.int32),
            pltpu.VMEM((_PER_W,), jnp.int32),
            pltpu.VMEM((_C, _D), jnp.float32),
            pltpu.VMEM((_C, _D), jnp.float32),
            pltpu.VMEM((_C, _D), jnp.float32),
            pltpu.VMEM((_C, _D), jnp.float32),
            pltpu.VMEM((_C, _D), jnp.float32),
            pltpu.VMEM((_C, _D), jnp.float32),
            pltpu.SemaphoreType.DMA,
            pltpu.SemaphoreType.DMA,
            pltpu.SemaphoreType.DMA,
            pltpu.SemaphoreType.DMA,
        ],
    )


_LN_BLK = 4096
_NPAIR = _N // 2


def _ln_half(x, g, b):
    mean = jnp.mean(x, axis=-1, keepdims=True)
    cent = x - mean
    var = jnp.mean(cent * cent, axis=-1, keepdims=True)
    return cent * lax.rsqrt(var + 1e-5) * g + b


def _ln_body(x_ref, g_ref, b_ref, o_ref):
    x = x_ref[...]
    g = g_ref[...]
    b = b_ref[...]
    ya = _ln_half(x[:, :_D], g, b)
    yb = _ln_half(x[:, _D:], g, b)
    o_ref[...] = jnp.concatenate([ya, yb], axis=-1)


def _layernorm(x2, ln_g, ln_b):
    grid = (_NPAIR // _LN_BLK,)
    return pl.pallas_call(
        _ln_body,
        grid=grid,
        in_specs=[
            pl.BlockSpec((_LN_BLK, 2 * _D), lambda g: (g, 0)),
            pl.BlockSpec((1, _D), lambda g: (0, 0)),
            pl.BlockSpec((1, _D), lambda g: (0, 0)),
        ],
        out_specs=pl.BlockSpec((_LN_BLK, 2 * _D), lambda g: (g, 0)),
        out_shape=jax.ShapeDtypeStruct((_NPAIR, 2 * _D), jnp.float32),
    )(x2, ln_g.reshape(1, _D), ln_b.reshape(1, _D))


def kernel(midi_events, event_types, note_table, pos_enc, special_dur, pos_w,
           pos_b, vel_w, vel_b, type_table, ln_g, ln_b):
    pos90 = pos_enc[:90]
    t1_3d, t2_4d = _build_tables(pos90, note_table, pos_w, pos_b, special_dur,
                                 type_table, vel_w.reshape(1, _D), vel_b)
    t1 = t1_3d.reshape(90 * 90, _D)
    t2 = t2_4d.reshape(2 * 90 * 90, _D)
    meT = midi_events.reshape(_N, 4).T.reshape(4, _IDX_ROWS, _IDX_BLK)
    et3 = event_types.reshape(1, _IDX_ROWS, _IDX_BLK)
    i1_2, i2_2 = _build_idx(meT, et3)
    i1 = i1_2.reshape(_N)
    i2 = i2_2.reshape(_N)
    comb = _sc_main()(i1, i2, t1, t2)
    comb2 = comb.reshape(_NPAIR, 2 * _D)
    out = _layernorm(comb2, ln_g, ln_b)
    return out.reshape(_B, _SEQ, _D)


# speedup vs baseline: 9.5069x; 1.1638x over previous
import functools

import jax
import jax.numpy as jnp
from jax import lax
from jax.experimental import pallas as pl
from jax.experimental.pallas import tpu as pltpu
from jax.experimental.pallas import tpu_sc as plsc

_NC, _NS, _L = 2, 16, 16
_NW = _NC * _NS
_B, _SEQ, _D = 1024, 200, 64
_N = _B * _SEQ
_PER_W = _N // _NW
_C = 128
_CHUNKS = _PER_W // _C


def _tables_body(pos90_ref, note_ref, posw_ref, posb_ref, sdur_ref, ttab_ref,
                 velw_ref, velb_ref, t1_ref, t2_ref):
    pos90 = pos90_ref[...]
    d2 = lax.dot_general(pos90, posw_ref[...],
                         (((1,), (1,)), ((), ())),
                         preferred_element_type=jnp.float32)
    d2 = d2 + posb_ref[...]
    row = lax.broadcasted_iota(jnp.int32, (90, 1), 0)
    d2 = jnp.where(row == 0, sdur_ref[...], d2)
    t1_ref[...] = pos90[:, None, :] + note_ref[...][None, :, :]
    vel = row.astype(jnp.float32) / 10.0
    velrows = vel * velw_ref[...] + velb_ref[...]
    t2_ref[...] = (ttab_ref[...][:, None, None, :]
                   + d2[None, :, None, :]
                   + velrows[None, None, :, :])


def _build_tables(pos90, note_table, pos_w, pos_b, special_dur, type_table,
                  vel_wT, vel_b):
    return pl.pallas_call(
        _tables_body,
        out_shape=[
            jax.ShapeDtypeStruct((90, 90, _D), jnp.float32),
            jax.ShapeDtypeStruct((2, 90, 90, _D), jnp.float32),
        ],
    )(pos90, note_table, pos_w, pos_b.reshape(1, _D), special_dur,
      type_table, vel_wT, vel_b.reshape(1, _D))


_IDX_BLK = 8192
_IDX_ROWS = _N // _IDX_BLK


def _idx_body(me_ref, et_ref, i1_ref, i2_ref):
    x = me_ref[...]
    t = et_ref[0]
    i1_ref[...] = x[0] * 90 + x[1]
    i2_ref[...] = (t * 90 + x[2]) * 90 + x[3]


def _build_idx(meT, et3):
    return pl.pallas_call(
        _idx_body,
        out_shape=[
            jax.ShapeDtypeStruct((_IDX_ROWS, _IDX_BLK), jnp.int32),
            jax.ShapeDtypeStruct((_IDX_ROWS, _IDX_BLK), jnp.int32),
        ],
    )(meT, et3)


def _sc_body(i1_hbm, i2_hbm, t1_hbm, t2_hbm, out_hbm,
             i1_v, i2_v, b1_a, b1_b, b2_a, b2_b, o_a, o_b,
             gsem_a, gsem_b, osem_a, osem_b):
    wid = lax.axis_index("s") * _NC + lax.axis_index("c")
    base0 = wid * _PER_W
    pltpu.sync_copy(i1_hbm.at[pl.ds(base0, _PER_W)], i1_v)
    pltpu.sync_copy(i2_hbm.at[pl.ds(base0, _PER_W)], i2_v)

    sides = (
        (b1_a, b2_a, o_a, gsem_a, osem_a),
        (b1_b, b2_b, o_b, gsem_b, osem_b),
    )

    def fire_gathers(side, ci):
        b1_v, b2_v, _, gsem, _ = sides[side]
        pltpu.async_copy(t1_hbm.at[i1_v.at[pl.ds(ci * _C, _C)]], b1_v, gsem)
        pltpu.async_copy(t2_hbm.at[i2_v.at[pl.ds(ci * _C, _C)]], b2_v, gsem)

    def process(side, oi):
        b1_v, b2_v, o_v, gsem, osem = sides[side]
        ci = 2 * oi + side
        head = i1_v.at[pl.ds(0, _C)]
        pltpu.make_async_copy(t1_hbm.at[head], b1_v, gsem).wait()
        pltpu.make_async_copy(t2_hbm.at[head], b2_v, gsem).wait()
        @pl.when(ci >= 2)
        def _():
            pltpu.make_async_copy(o_v, out_hbm.at[pl.ds(base0, _C)],
                                  osem).wait()

        def tok(ti, carry):
            t0 = ti * 8
            for u in range(8):
                for k in range(4):
                    sl = pl.ds(_L * k, _L)
                    o_v[t0 + u, sl] = b1_v[t0 + u, sl] + b2_v[t0 + u, sl]
            return carry

        lax.fori_loop(0, _C // 8, tok, 0)
        pltpu.async_copy(o_v, out_hbm.at[pl.ds(base0 + ci * _C, _C)], osem)

        @pl.when(ci + 2 < _CHUNKS)
        def _():
            fire_gathers(side, ci + 2)

    fire_gathers(0, 0)
    fire_gathers(1, 1)

    def outer(oi, carry):
        process(0, oi)
        process(1, oi)
        return carry

    lax.fori_loop(0, _CHUNKS // 2, outer, 0)
    for _, _, o_v, _, osem in sides:
        pltpu.make_async_copy(o_v, out_hbm.at[pl.ds(base0, _C)], osem).wait()


@functools.cache
def _sc_main():
    return pl.kernel(
        _sc_body,
        out_type=jax.ShapeDtypeStruct((_N, _D), jnp.float32),
        mesh=plsc.VectorSubcoreMesh(core_axis_name="c", subcore_axis_name="s",
                                    num_cores=_NC, num_subcores=_NS),
        compiler_params=pltpu.CompilerParams(use_tc_tiling_on_sc=False),
        scratch_types=[
            pltpu.VMEM((_PER_W,), jnp.int32),
            pltpu.VMEM((_PER_W,), jnp.int32),
            pltpu.VMEM((_C, _D), jnp.float32),
            pltpu.VMEM((_C, _D), jnp.float32),
            pltpu.VMEM((_C, _D), jnp.float32),
            pltpu.VMEM((_C, _D), jnp.float32),
            pltpu.VMEM((_C, _D), jnp.float32),
            pltpu.VMEM((_C, _D), jnp.float32),
            pltpu.SemaphoreType.DMA,
            pltpu.SemaphoreType.DMA,
            pltpu.SemaphoreType.DMA,
            pltpu.SemaphoreType.DMA,
        ],
    )


_LN_BLK = 4096
_NPAIR = _N // 2


def _ln_body(x_ref, m_ref, p_ref, g_ref, b_ref, o_ref):
    x = x_ref[...]
    mm = m_ref[...]
    pp = p_ref[...]
    dot = functools.partial(lax.dot_general,
                            dimension_numbers=(((1,), (0,)), ((), ())),
                            preferred_element_type=jnp.float32)
    m2 = dot(x, mm)
    q2 = dot(x * x, mm)
    bm = dot(m2, pp)
    bq = dot(q2, pp)
    var = bq - bm * bm
    o_ref[...] = (x - bm) * lax.rsqrt(var + 1e-5) * g_ref[...] + b_ref[...]


def _layernorm(x2, ln_g, ln_b):
    half = jnp.concatenate([jnp.ones((_D,), jnp.float32) / _D,
                            jnp.zeros((_D,), jnp.float32)])
    mm = jnp.zeros((2 * _D, 8), jnp.float32)
    mm = mm.at[:, 0].set(half).at[:, 1].set(half[::-1])
    pp = jnp.zeros((8, 2 * _D), jnp.float32)
    pp = pp.at[0].set(half * _D).at[1].set(half[::-1] * _D)
    g2 = jnp.tile(ln_g, 2).reshape(1, 2 * _D)
    b2 = jnp.tile(ln_b, 2).reshape(1, 2 * _D)
    grid = (_NPAIR // _LN_BLK,)
    return pl.pallas_call(
        _ln_body,
        grid=grid,
        in_specs=[
            pl.BlockSpec((_LN_BLK, 2 * _D), lambda g: (g, 0)),
            pl.BlockSpec((2 * _D, 8), lambda g: (0, 0)),
            pl.BlockSpec((8, 2 * _D), lambda g: (0, 0)),
            pl.BlockSpec((1, 2 * _D), lambda g: (0, 0)),
            pl.BlockSpec((1, 2 * _D), lambda g: (0, 0)),
        ],
        out_specs=pl.BlockSpec((_LN_BLK, 2 * _D), lambda g: (g, 0)),
        out_shape=jax.ShapeDtypeStruct((_NPAIR, 2 * _D), jnp.float32),
    )(x2, mm, pp, g2, b2)


def kernel(midi_events, event_types, note_table, pos_enc, special_dur, pos_w,
           pos_b, vel_w, vel_b, type_table, ln_g, ln_b):
    pos90 = pos_enc[:90]
    t1_3d, t2_4d = _build_tables(pos90, note_table, pos_w, pos_b, special_dur,
                                 type_table, vel_w.reshape(1, _D), vel_b)
    t1 = t1_3d.reshape(90 * 90, _D)
    t2 = t2_4d.reshape(2 * 90 * 90, _D)
    meT = midi_events.reshape(_N, 4).T.reshape(4, _IDX_ROWS, _IDX_BLK)
    et3 = event_types.reshape(1, _IDX_ROWS, _IDX_BLK)
    i1_2, i2_2 = _build_idx(meT, et3)
    i1 = i1_2.reshape(_N)
    i2 = i2_2.reshape(_N)
    comb = _sc_main()(i1, i2, t1, t2)
    comb2 = comb.reshape(_NPAIR, 2 * _D)
    out2 = _layernorm(comb2, ln_g, ln_b)
    ldb = jnp.transpose(out2.reshape(_B, _SEQ, _D), (1, 2, 0))
    ldb = lax.optimization_barrier(ldb)
    return jnp.transpose(ldb, (2, 0, 1))


# speedup vs baseline: 9.5185x; 1.0012x over previous
import functools

import jax
import jax.numpy as jnp
from jax import lax
from jax.experimental import pallas as pl
from jax.experimental.pallas import tpu as pltpu
from jax.experimental.pallas import tpu_sc as plsc

_NC, _NS, _L = 2, 16, 16
_NW = _NC * _NS
_B, _SEQ, _D = 1024, 200, 64
_N = _B * _SEQ
_PER_W = _N // _NW
_C = 128
_CHUNKS = _PER_W // _C


def _tables_body(pos90_ref, note_ref, posw_ref, posb_ref, sdur_ref, ttab_ref,
                 velw_ref, velb_ref, t1_ref, t2_ref):
    pos90 = pos90_ref[...]
    d2 = lax.dot_general(pos90, posw_ref[...],
                         (((1,), (1,)), ((), ())),
                         preferred_element_type=jnp.float32)
    d2 = d2 + posb_ref[...]
    row = lax.broadcasted_iota(jnp.int32, (90, 1), 0)
    d2 = jnp.where(row == 0, sdur_ref[...], d2)
    t1_ref[...] = pos90[:, None, :] + note_ref[...][None, :, :]
    vel = row.astype(jnp.float32) / 10.0
    velrows = vel * velw_ref[...] + velb_ref[...]
    t2_ref[...] = (ttab_ref[...][:, None, None, :]
                   + d2[None, :, None, :]
                   + velrows[None, None, :, :])


def _build_tables(pos90, note_table, pos_w, pos_b, special_dur, type_table,
                  vel_wT, vel_b):
    return pl.pallas_call(
        _tables_body,
        out_shape=[
            jax.ShapeDtypeStruct((90, 90, _D), jnp.float32),
            jax.ShapeDtypeStruct((2, 90, 90, _D), jnp.float32),
        ],
    )(pos90, note_table, pos_w, pos_b.reshape(1, _D), special_dur,
      type_table, vel_wT, vel_b.reshape(1, _D))


_IDX_BLK = 8192
_IDX_ROWS = _N // _IDX_BLK


def _idx_body(me_ref, et_ref, i1_ref, i2_ref):
    x = me_ref[...]
    t = et_ref[0]
    i1_ref[...] = x[0] * 90 + x[1]
    i2_ref[...] = (t * 90 + x[2]) * 90 + x[3]


def _build_idx(meT, et3):
    return pl.pallas_call(
        _idx_body,
        out_shape=[
            jax.ShapeDtypeStruct((_IDX_ROWS, _IDX_BLK), jnp.int32),
            jax.ShapeDtypeStruct((_IDX_ROWS, _IDX_BLK), jnp.int32),
        ],
    )(meT, et3)


def _sc_body(i1_hbm, i2_hbm, t1_hbm, t2_hbm, out_hbm,
             i1_v, i2_v, b1_a, b1_b, b2_a, b2_b, o_a, o_b,
             gsem_a, gsem_b, osem_a, osem_b):
    wid = lax.axis_index("s") * _NC + lax.axis_index("c")
    base0 = wid * _PER_W
    pltpu.sync_copy(i1_hbm.at[pl.ds(base0, _PER_W)], i1_v)
    pltpu.sync_copy(i2_hbm.at[pl.ds(base0, _PER_W)], i2_v)

    sides = (
        (b1_a, b2_a, o_a, gsem_a, osem_a),
        (b1_b, b2_b, o_b, gsem_b, osem_b),
    )

    def fire_gathers(side, ci):
        b1_v, b2_v, _, gsem, _ = sides[side]
        pltpu.async_copy(t1_hbm.at[i1_v.at[pl.ds(ci * _C, _C)]], b1_v, gsem)
        pltpu.async_copy(t2_hbm.at[i2_v.at[pl.ds(ci * _C, _C)]], b2_v, gsem)

    def process(side, oi):
        b1_v, b2_v, o_v, gsem, osem = sides[side]
        ci = 2 * oi + side
        head = i1_v.at[pl.ds(0, _C)]
        pltpu.make_async_copy(t1_hbm.at[head], b1_v, gsem).wait()
        pltpu.make_async_copy(t2_hbm.at[head], b2_v, gsem).wait()
        @pl.when(ci >= 2)
        def _():
            pltpu.make_async_copy(o_v, out_hbm.at[pl.ds(base0, _C)],
                                  osem).wait()

        def tok(ti, carry):
            t0 = ti * 8
            for u in range(8):
                for k in range(4):
                    sl = pl.ds(_L * k, _L)
                    o_v[t0 + u, sl] = b1_v[t0 + u, sl] + b2_v[t0 + u, sl]
            return carry

        lax.fori_loop(0, _C // 8, tok, 0)
        pltpu.async_copy(o_v, out_hbm.at[pl.ds(base0 + ci * _C, _C)], osem)

        @pl.when(ci + 2 < _CHUNKS)
        def _():
            fire_gathers(side, ci + 2)

    fire_gathers(0, 0)
    fire_gathers(1, 1)

    def outer(oi, carry):
        process(0, oi)
        process(1, oi)
        return carry

    lax.fori_loop(0, _CHUNKS // 2, outer, 0)
    for _, _, o_v, _, osem in sides:
        pltpu.make_async_copy(o_v, out_hbm.at[pl.ds(base0, _C)], osem).wait()


@functools.cache
def _sc_main():
    return pl.kernel(
        _sc_body,
        out_type=jax.ShapeDtypeStruct((_N, _D), jnp.float32),
        mesh=plsc.VectorSubcoreMesh(core_axis_name="c", subcore_axis_name="s",
                                    num_cores=_NC, num_subcores=_NS),
        compiler_params=pltpu.CompilerParams(use_tc_tiling_on_sc=False),
        scratch_types=[
            pltpu.VMEM((_PER_W,), jnp.int32),
            pltpu.VMEM((_PER_W,), jnp.int32),
            pltpu.VMEM((_C, _D), jnp.float32),
            pltpu.VMEM((_C, _D), jnp.float32),
            pltpu.VMEM((_C, _D), jnp.float32),
            pltpu.VMEM((_C, _D), jnp.float32),
            pltpu.VMEM((_C, _D), jnp.float32),
            pltpu.VMEM((_C, _D), jnp.float32),
            pltpu.SemaphoreType.DMA,
            pltpu.SemaphoreType.DMA,
            pltpu.SemaphoreType.DMA,
            pltpu.SemaphoreType.DMA,
        ],
    )


_LN_BLK = 4096
_NPAIR = _N // 2


def _ln_body(x_ref, m_ref, p_ref, g_ref, b_ref, o_ref):
    x = x_ref[...]
    mm = m_ref[...]
    pp = p_ref[...]
    dot = functools.partial(lax.dot_general,
                            dimension_numbers=(((1,), (0,)), ((), ())),
                            preferred_element_type=jnp.float32)
    m2 = dot(x, mm)
    q2 = dot(x * x, mm)
    bm = dot(m2, pp)
    bq = dot(q2, pp)
    var = bq - bm * bm
    o_ref[...] = (x - bm) * lax.rsqrt(var + 1e-5) * g_ref[...] + b_ref[...]


def _layernorm(x2, ln_g, ln_b):
    half = jnp.concatenate([jnp.ones((_D,), jnp.float32) / _D,
                            jnp.zeros((_D,), jnp.float32)])
    mm = jnp.zeros((2 * _D, 8), jnp.float32)
    mm = mm.at[:, 0].set(half).at[:, 1].set(half[::-1])
    pp = jnp.zeros((8, 2 * _D), jnp.float32)
    pp = pp.at[0].set(half * _D).at[1].set(half[::-1] * _D)
    g2 = jnp.tile(ln_g, 2).reshape(1, 2 * _D)
    b2 = jnp.tile(ln_b, 2).reshape(1, 2 * _D)
    grid = (_NPAIR // _LN_BLK,)
    return pl.pallas_call(
        _ln_body,
        grid=grid,
        in_specs=[
            pl.BlockSpec((_LN_BLK, 2 * _D), lambda g: (g, 0)),
            pl.BlockSpec((2 * _D, 8), lambda g: (0, 0)),
            pl.BlockSpec((8, 2 * _D), lambda g: (0, 0)),
            pl.BlockSpec((1, 2 * _D), lambda g: (0, 0)),
            pl.BlockSpec((1, 2 * _D), lambda g: (0, 0)),
        ],
        out_specs=pl.BlockSpec((_LN_BLK, 2 * _D), lambda g: (g, 0)),
        out_shape=jax.ShapeDtypeStruct((_NPAIR, 2 * _D), jnp.float32),
    )(x2, mm, pp, g2, b2)


def kernel(midi_events, event_types, note_table, pos_enc, special_dur, pos_w,
           pos_b, vel_w, vel_b, type_table, ln_g, ln_b):
    pos90 = pos_enc[:90]
    t1_3d, t2_4d = _build_tables(pos90, note_table, pos_w, pos_b, special_dur,
                                 type_table, vel_w.reshape(1, _D), vel_b)
    t1 = t1_3d.reshape(90 * 90, _D)
    t2 = t2_4d.reshape(2 * 90 * 90, _D)
    meT = midi_events.reshape(_N, 4).T.reshape(4, _IDX_ROWS, _IDX_BLK)
    et3 = event_types.reshape(1, _IDX_ROWS, _IDX_BLK)
    i1_2, i2_2 = _build_idx(meT, et3)
    i1 = i1_2.reshape(_N)
    i2 = i2_2.reshape(_N)
    comb = _sc_main()(i1, i2, t1, t2)
    comb2 = comb.reshape(_NPAIR, 2 * _D)
    out2 = _layernorm(comb2, ln_g, ln_b)
    return out2.reshape(_B, _SEQ, _D)
